# baseline (speedup 1.0000x reference)
import functools

import jax
import jax.numpy as jnp
from jax import lax
from jax.experimental import pallas as pl
from jax.experimental.pallas import tpu as pltpu
from jax.experimental.pallas import tpu_sc as plsc

_NW = 32
_CH = 32
_IW = 128
_L = 16


def kernel(numerical, categorical, tables):
    B, ND = numerical.shape
    F, V, D = tables.shape
    OUT_D = ND + F * D
    bpw = B // _NW
    nch = bpw // _CH
    niw = bpw // _IW

    cat_w = categorical.T.reshape(F, _NW, niw, _IW)
    tab_g = tables.reshape(F, V // 8, 8, D)
    dummy = jnp.zeros((_CH * 8, D), jnp.float32)

    mesh = plsc.VectorSubcoreMesh(core_axis_name="c", subcore_axis_name="s")

    @functools.partial(
        pl.kernel,
        out_type=jax.ShapeDtypeStruct((B, OUT_D), jnp.float32),
        mesh=mesh,
        compiler_params=pltpu.CompilerParams(needs_layout_passes=False),
        scratch_types=[
            pltpu.VMEM((F, niw, _IW), jnp.int32),
            pltpu.VMEM((_CH * 8, D), jnp.float32),
            pltpu.VMEM((_CH * 8, D), jnp.float32),
            pltpu.VMEM((_CH, ND), jnp.float32),
            pltpu.VMEM((_CH, OUT_D), jnp.float32),
            pltpu.SemaphoreType.DMA,
            pltpu.SemaphoreType.DMA,
        ],
    )
    def enc(num_hbm, cat_hbm, tab_hbm, dum_hbm, out_hbm, idx_v, grp_a,
            grp_b, num_v, big_v, sem_a, sem_b):
        wid = lax.axis_index("s") * 2 + lax.axis_index("c")
        base = wid * bpw
        lane = lax.iota(jnp.int32, _L)

        pltpu.sync_copy(cat_hbm.at[:, wid], idx_v)

        def fire(j, f, grp, sem):
            for g in range(_CH // _L):
                i0 = j * _CH + g * _L
                idx16 = idx_v[f, i0 // _IW, pl.ds(i0 % _IW, _L)]
                gid16 = lax.shift_right_logical(idx16, 3)
                for k in range(_L):
                    pltpu.async_copy(
                        tab_hbm.at[f, gid16[k]],
                        grp.at[pl.ds((g * _L + k) * 8, 8)], sem)

        def drain(grp, sem):
            pltpu.make_async_copy(
                dum_hbm, grp, sem
            ).wait()

        def extract(j, f, grp):
            fcol = jnp.full((_L,), ND + f * D, jnp.int32)
            for g in range(_CH // _L):
                i0 = j * _CH + g * _L
                idx16 = idx_v[f, i0 // _IW, pl.ds(i0 % _IW, _L)]
                rv = lax.bitwise_and(idx16, jnp.full((_L,), 7, jnp.int32))
                jv = g * _L + lane
                rowb = jv * 8 + rv
                for c in range(D):
                    cv = jnp.full((_L,), c, jnp.int32)
                    vals = plsc.load_gather(grp, [rowb, cv])
                    plsc.store_scatter(big_v, [jv, fcol + cv], vals)

        def chunk_body(j, carry):
            b0 = pl.multiple_of(base + j * _CH, _CH)
            pltpu.sync_copy(num_hbm.at[pl.ds(b0, _CH), :], num_v)
            for g in range(_CH // _L):
                jv = g * _L + lane
                for c in range(ND):
                    cv = jnp.full((_L,), c, jnp.int32)
                    vals = plsc.load_gather(num_v, [jv, cv])
                    plsc.store_scatter(big_v, [jv, cv], vals)

            fire(j, 0, grp_a, sem_a)

            def pair_body(t, carry2):
                f0 = t * 2
                fire(j, f0 + 1, grp_b, sem_b)
                drain(grp_a, sem_a)
                extract(j, f0, grp_a)

                @pl.when(f0 + 2 < F)
                def _():
                    fire(j, f0 + 2, grp_a, sem_a)

                drain(grp_b, sem_b)
                extract(j, f0 + 1, grp_b)
                return carry2

            lax.fori_loop(0, F // 2, pair_body, 0)
            pltpu.sync_copy(big_v, out_hbm.at[pl.ds(b0, _CH), :])
            return carry

        lax.fori_loop(0, nch, chunk_body, 0)

    return enc(numerical, cat_w, tab_g, dummy)


# speedup vs baseline: 1.0380x; 1.0380x over previous
import functools

import jax
import jax.numpy as jnp
from jax import lax
from jax.experimental import pallas as pl
from jax.experimental.pallas import tpu as pltpu
from jax.experimental.pallas import tpu_sc as plsc

_NW = 32
_CH = 32
_IW = 128
_L = 16


def kernel(numerical, categorical, tables):
    B, ND = numerical.shape
    F, V, D = tables.shape
    OUT_D = ND + F * D
    bpw = B // _NW
    nch = bpw // _CH
    niw = bpw // _IW

    cat_w = categorical.T.reshape(F, _NW, niw, _IW)
    tab_g = tables.reshape(F, V // 8, 8, D)
    dummy = jnp.zeros((_CH * 8, D), jnp.float32)

    mesh = plsc.VectorSubcoreMesh(core_axis_name="c", subcore_axis_name="s")

    @functools.partial(
        pl.kernel,
        out_type=jax.ShapeDtypeStruct((B, OUT_D), jnp.float32),
        mesh=mesh,
        compiler_params=pltpu.CompilerParams(needs_layout_passes=False),
        scratch_types=[
            pltpu.VMEM((F, niw, _IW), jnp.int32),
            pltpu.VMEM((_CH * 8, D), jnp.float32),
            pltpu.VMEM((_CH * 8, D), jnp.float32),
            pltpu.VMEM((_CH, ND), jnp.float32),
            pltpu.VMEM((_CH, OUT_D), jnp.float32),
            pltpu.SemaphoreType.DMA,
            pltpu.SemaphoreType.DMA,
            pltpu.SemaphoreType.DMA,
        ],
    )
    def enc(num_hbm, cat_hbm, tab_hbm, dum_hbm, out_hbm, idx_v, grp_a,
            grp_b, num_v, big_v, sem_a, sem_b, sem_w):
        wid = lax.axis_index("s") * 2 + lax.axis_index("c")
        base = wid * bpw
        lane = lax.iota(jnp.int32, _L)

        pltpu.sync_copy(cat_hbm.at[:, wid], idx_v)

        def fire(j, f, grp, sem):
            for g in range(_CH // _L):
                i0 = j * _CH + g * _L
                idx16 = idx_v[f, i0 // _IW, pl.ds(i0 % _IW, _L)]
                gid16 = lax.shift_right_logical(idx16, 3)
                for k in range(_L):
                    pltpu.async_copy(
                        tab_hbm.at[f, gid16[k]],
                        grp.at[pl.ds((g * _L + k) * 8, 8)], sem)

        def drain(grp, sem):
            pltpu.make_async_copy(
                dum_hbm, grp, sem
            ).wait()

        def extract(j, f, grp):
            fcol = jnp.full((_L,), ND + f * D, jnp.int32)
            for g in range(_CH // _L):
                i0 = j * _CH + g * _L
                idx16 = idx_v[f, i0 // _IW, pl.ds(i0 % _IW, _L)]
                rv = lax.bitwise_and(idx16, jnp.full((_L,), 7, jnp.int32))
                jv = g * _L + lane
                rowb = jv * 8 + rv
                for c in range(D):
                    cv = jnp.full((_L,), c, jnp.int32)
                    vals = plsc.load_gather(grp, [rowb, cv])
                    plsc.store_scatter(big_v, [jv, fcol + cv], vals)

        def chunk_body(j, carry):
            b0 = pl.multiple_of(base + j * _CH, _CH)
            fire(j, 0, grp_a, sem_a)
            pltpu.sync_copy(num_hbm.at[pl.ds(b0, _CH), :], num_v)

            @pl.when(j > 0)
            def _():
                pltpu.make_async_copy(
                    big_v, out_hbm.at[pl.ds(0, _CH), :], sem_w).wait()

            for g in range(_CH // _L):
                jv = g * _L + lane
                for c in range(ND):
                    cv = jnp.full((_L,), c, jnp.int32)
                    vals = plsc.load_gather(num_v, [jv, cv])
                    plsc.store_scatter(big_v, [jv, cv], vals)

            def pair_body(t, carry2):
                f0 = t * 2
                fire(j, f0 + 1, grp_b, sem_b)
                drain(grp_a, sem_a)
                extract(j, f0, grp_a)

                @pl.when(f0 + 2 < F)
                def _():
                    fire(j, f0 + 2, grp_a, sem_a)

                drain(grp_b, sem_b)
                extract(j, f0 + 1, grp_b)
                return carry2

            lax.fori_loop(0, F // 2, pair_body, 0)
            pltpu.async_copy(big_v, out_hbm.at[pl.ds(b0, _CH), :], sem_w)
            return carry

        lax.fori_loop(0, nch, chunk_body, 0)
        pltpu.make_async_copy(
            big_v, out_hbm.at[pl.ds(0, _CH), :], sem_w).wait()

    return enc(numerical, cat_w, tab_g, dummy)


# speedup vs baseline: 1.0716x; 1.0323x over previous
import functools

import jax
import jax.numpy as jnp
from jax import lax
from jax.experimental import pallas as pl
from jax.experimental.pallas import tpu as pltpu
from jax.experimental.pallas import tpu_sc as plsc

_NW = 32
_CH = 32
_IW = 128
_L = 16


def kernel(numerical, categorical, tables):
    B, ND = numerical.shape
    F, V, D = tables.shape
    OUT_D = ND + F * D
    bpw = B // _NW
    nch = bpw // _CH
    niw = bpw // _IW

    cat_w = categorical.T.reshape(F, _NW, niw, _IW)
    tab_g = tables.reshape(F, V // 8, 8, D)
    dummy = jnp.zeros((_CH * 8, D), jnp.float32)

    mesh = plsc.VectorSubcoreMesh(core_axis_name="c", subcore_axis_name="s")

    @functools.partial(
        pl.kernel,
        out_type=jax.ShapeDtypeStruct((B, OUT_D), jnp.float32),
        mesh=mesh,
        compiler_params=pltpu.CompilerParams(needs_layout_passes=False),
        scratch_types=[
            pltpu.VMEM((F, niw, _IW), jnp.int32),
            pltpu.VMEM((_CH * 8, D), jnp.float32),
            pltpu.VMEM((_CH * 8, D), jnp.float32),
            pltpu.VMEM((_CH, ND), jnp.float32),
            pltpu.VMEM((_CH, OUT_D), jnp.float32),
            pltpu.SemaphoreType.DMA,
            pltpu.SemaphoreType.DMA,
            pltpu.SemaphoreType.DMA,
        ],
    )
    def enc(num_hbm, cat_hbm, tab_hbm, dum_hbm, out_hbm, idx_v, grp_a,
            grp_b, num_v, big_v, sem_a, sem_b, sem_w):
        wid = lax.axis_index("s") * 2 + lax.axis_index("c")
        base = wid * bpw
        lane = lax.iota(jnp.int32, _L)

        pltpu.sync_copy(cat_hbm.at[:, wid], idx_v)

        def fire(j, f, grp, sem):
            for g in range(_CH // _L):
                i0 = j * _CH + g * _L
                idx16 = idx_v[f, i0 // _IW, pl.ds(i0 % _IW, _L)]
                gid16 = lax.shift_right_logical(idx16, 3)
                for k in range(_L):
                    pltpu.async_copy(
                        tab_hbm.at[f, gid16[k]],
                        grp.at[pl.ds((g * _L + k) * 8, 8)], sem)

        def drain(grp, sem):
            pltpu.make_async_copy(
                dum_hbm, grp, sem
            ).wait()

        def extract(j, f, grp):
            col0 = ND + f * D
            for g in range(_CH // _L):
                i0 = j * _CH + g * _L
                idx16 = idx_v[f, i0 // _IW, pl.ds(i0 % _IW, _L)]
                rv = lax.bitwise_and(idx16, jnp.full((_L,), 7, jnp.int32))
                for k in range(_L):
                    jrow = g * _L + k
                    rowb = jrow * 8 + rv[k]
                    for c0 in (0, _L, D - _L):
                        big_v[jrow, pl.ds(col0 + c0, _L)] = (
                            grp[rowb, pl.ds(c0, _L)])

        def chunk_body(j, carry):
            b0 = pl.multiple_of(base + j * _CH, _CH)
            fire(j, 0, grp_a, sem_a)
            pltpu.sync_copy(num_hbm.at[pl.ds(b0, _CH), :], num_v)

            @pl.when(j > 0)
            def _():
                pltpu.make_async_copy(
                    big_v, out_hbm.at[pl.ds(0, _CH), :], sem_w).wait()

            for g in range(_CH // _L):
                jv = g * _L + lane
                for c in range(ND):
                    cv = jnp.full((_L,), c, jnp.int32)
                    vals = plsc.load_gather(num_v, [jv, cv])
                    plsc.store_scatter(big_v, [jv, cv], vals)

            def pair_body(t, carry2):
                f0 = t * 2
                fire(j, f0 + 1, grp_b, sem_b)
                drain(grp_a, sem_a)
                extract(j, f0, grp_a)

                @pl.when(f0 + 2 < F)
                def _():
                    fire(j, f0 + 2, grp_a, sem_a)

                drain(grp_b, sem_b)
                extract(j, f0 + 1, grp_b)
                return carry2

            lax.fori_loop(0, F // 2, pair_body, 0)
            pltpu.async_copy(big_v, out_hbm.at[pl.ds(b0, _CH), :], sem_w)
            return carry

        lax.fori_loop(0, nch, chunk_body, 0)
        pltpu.make_async_copy(
            big_v, out_hbm.at[pl.ds(0, _CH), :], sem_w).wait()

    return enc(numerical, cat_w, tab_g, dummy)


# speedup vs baseline: 1.0982x; 1.0249x over previous
import functools

import jax
import jax.numpy as jnp
from jax import lax
from jax.experimental import pallas as pl
from jax.experimental.pallas import tpu as pltpu
from jax.experimental.pallas import tpu_sc as plsc

_NW = 32
_CH = 32
_IW = 128
_L = 16


def kernel(numerical, categorical, tables):
    B, ND = numerical.shape
    F, V, D = tables.shape
    OUT_D = ND + F * D
    bpw = B // _NW
    nch = bpw // _CH
    niw = bpw // _IW

    cat_w = categorical.T.reshape(F, _NW, niw, _IW)
    tab_g = tables.reshape(F, V // 8, 8, D)
    dummy = jnp.zeros((_CH * 8, D), jnp.float32)

    mesh = plsc.VectorSubcoreMesh(core_axis_name="c", subcore_axis_name="s")

    @functools.partial(
        pl.kernel,
        out_type=jax.ShapeDtypeStruct((B, OUT_D), jnp.float32),
        mesh=mesh,
        compiler_params=pltpu.CompilerParams(needs_layout_passes=False),
        scratch_types=[
            pltpu.VMEM((F, niw, _IW), jnp.int32),
            pltpu.VMEM((_CH * 8, D), jnp.float32),
            pltpu.VMEM((_CH * 8, D), jnp.float32),
            pltpu.VMEM((_CH, ND), jnp.float32),
            pltpu.VMEM((_CH, OUT_D), jnp.float32),
            pltpu.SemaphoreType.DMA,
            pltpu.SemaphoreType.DMA,
            pltpu.SemaphoreType.DMA,
        ],
    )
    def enc(num_hbm, cat_hbm, tab_hbm, dum_hbm, out_hbm, idx_v, grp_a,
            grp_b, num_v, big_v, sem_a, sem_b, sem_w):
        wid = lax.axis_index("s") * 2 + lax.axis_index("c")
        base = wid * bpw
        lane = lax.iota(jnp.int32, _L)

        pltpu.sync_copy(cat_hbm.at[:, wid], idx_v)

        def fire(j, f, grp, sem):
            for g in range(_CH // _L):
                i0 = j * _CH + g * _L
                idx16 = idx_v[f, i0 // _IW, pl.ds(i0 % _IW, _L)]
                gid16 = lax.shift_right_logical(idx16, 3)
                for k in range(_L):
                    pltpu.async_copy(
                        tab_hbm.at[f, gid16[k]],
                        grp.at[pl.ds((g * _L + k) * 8, 8)], sem)

        def drain_extract(j, f, grp, sem):
            fcol = jnp.full((_L,), ND + f * D, jnp.int32)
            for g in range(_CH // _L):
                pltpu.make_async_copy(
                    dum_hbm.at[pl.ds(0, _L * 8), :],
                    grp.at[pl.ds(g * _L * 8, _L * 8)], sem).wait()
                i0 = j * _CH + g * _L
                idx16 = idx_v[f, i0 // _IW, pl.ds(i0 % _IW, _L)]
                rv = lax.bitwise_and(idx16, jnp.full((_L,), 7, jnp.int32))
                jv = g * _L + lane
                rowb = jv * 8 + rv
                for c in range(D):
                    cv = jnp.full((_L,), c, jnp.int32)
                    vals = plsc.load_gather(grp, [rowb, cv])
                    plsc.store_scatter(big_v, [jv, fcol + cv], vals)

        def chunk_body(j, carry):
            b0 = pl.multiple_of(base + j * _CH, _CH)
            fire(j, 0, grp_a, sem_a)
            pltpu.sync_copy(num_hbm.at[pl.ds(b0, _CH), :], num_v)

            @pl.when(j > 0)
            def _():
                pltpu.make_async_copy(
                    big_v, out_hbm.at[pl.ds(0, _CH), :], sem_w).wait()

            for g in range(_CH // _L):
                jv = g * _L + lane
                for c in range(ND):
                    cv = jnp.full((_L,), c, jnp.int32)
                    vals = plsc.load_gather(num_v, [jv, cv])
                    plsc.store_scatter(big_v, [jv, cv], vals)

            def pair_body(t, carry2):
                f0 = t * 2
                fire(j, f0 + 1, grp_b, sem_b)
                drain_extract(j, f0, grp_a, sem_a)

                @pl.when(f0 + 2 < F)
                def _():
                    fire(j, f0 + 2, grp_a, sem_a)

                drain_extract(j, f0 + 1, grp_b, sem_b)
                return carry2

            lax.fori_loop(0, F // 2, pair_body, 0)
            pltpu.async_copy(big_v, out_hbm.at[pl.ds(b0, _CH), :], sem_w)
            return carry

        lax.fori_loop(0, nch, chunk_body, 0)
        pltpu.make_async_copy(
            big_v, out_hbm.at[pl.ds(0, _CH), :], sem_w).wait()

    return enc(numerical, cat_w, tab_g, dummy)
